# baseline (speedup 1.0000x reference)
import functools

import jax
import jax.numpy as jnp
from jax import lax
from jax.experimental import pallas as pl
from jax.experimental.pallas import tpu as pltpu
from jax.experimental.pallas import tpu_sc as plsc

V = 1000000
H = 128
T = 32
B = 4096
L = 200

_NC = 2
_NS = 16
_NW = _NC * _NS
_BPW = B // _NW
_LHALF = L // 2


_NB = 4
_NI = 8


def _pool_body(nrows, emb_hbm, idx_hbm, out_hbm, idx_v, rows_v, acc_v,
               sem0, sem1, sem2, sem3, isem):
    _BPW = nrows // _NW
    wid = lax.axis_index("s") * _NC + lax.axis_index("c")
    base = wid * _BPW

    sems = (sem0, sem1, sem2, sem3)

    def issue(b, buf, islot):
        sem = sems[buf]
        for j in range(2):
            pltpu.async_copy(
                emb_hbm.at[idx_v.at[islot, j]],
                rows_v.at[buf, j], sem)

    def drain(buf):
        pltpu.make_async_copy(
            emb_hbm.at[pl.ds(0, L)], rows_v.at[buf], sems[buf]).wait()

    def drain_idx(islot):
        pltpu.make_async_copy(
            idx_hbm.at[0], idx_v.at[islot], isem).wait()

    def accumulate(b, buf):
        def acc_body(r, accs):
            return tuple(
                accs[c]
                + rows_v[buf, 0, r, pl.ds(c * 16, 16)]
                + rows_v[buf, 1, r, pl.ds(c * 16, 16)]
                for c in range(H // 16)
            )

        accs = lax.fori_loop(
            0, _LHALF, acc_body,
            tuple(jnp.zeros((16,), jnp.float32) for _ in range(H // 16)),
            unroll=4,
        )
        for c in range(H // 16):
            acc_v[b, pl.ds(c * 16, 16)] = accs[c]

    for k in range(_NB):
        pltpu.sync_copy(idx_hbm.at[base + k], idx_v.at[k])
        issue(k, k, k)
    for k in range(_NB, _NI):
        pltpu.async_copy(idx_hbm.at[base + k], idx_v.at[k], isem)

    def quad_body(i, _):
        b = _NB * i
        for k in range(_NB):
            r = b + k
            drain(k)
            accumulate(r, k)

            @pl.when(r + _NB < _BPW)
            def _():
                drain_idx((r + _NB) % _NI)
                issue(r + _NB, k, (r + _NB) % _NI)

            @pl.when(r + _NI < _BPW)
            def _():
                pltpu.async_copy(
                    idx_hbm.at[base + r + _NI],
                    idx_v.at[(r + _NI) % _NI], isem)
        return 0

    lax.fori_loop(0, _BPW // _NB, quad_body, 0)

    pltpu.sync_copy(acc_v, out_hbm.at[pl.ds(base, _BPW)])


def _sc_pool(emb, idx3):
    nrows = idx3.shape[0]
    mesh = plsc.VectorSubcoreMesh(
        core_axis_name="c", subcore_axis_name="s",
        num_cores=_NC, num_subcores=_NS)
    k = pl.kernel(
        functools.partial(_pool_body, nrows),
        out_type=jax.ShapeDtypeStruct((nrows, H), jnp.float32),
        mesh=mesh,
        scratch_types=[
            pltpu.VMEM((_NI, 2, _LHALF), jnp.int32),
            pltpu.VMEM((_NB, 2, _LHALF, H), jnp.float32),
            pltpu.VMEM((nrows // _NW, H), jnp.float32),
            pltpu.SemaphoreType.DMA,
            pltpu.SemaphoreType.DMA,
            pltpu.SemaphoreType.DMA,
            pltpu.SemaphoreType.DMA,
            pltpu.SemaphoreType.DMA,
        ],
    )
    return k(emb, idx3)


def _mlp_body(pooled_ref, rd_ref, w1a_ref, w1b_ref, b1_ref, w2_ref, b2_ref,
              wmu_ref, cmu_ref, smu_ref, wlv_ref, clv_ref, slv_ref,
              mu_ref, sc_ref):
    rd = rd_ref[...]
    ave = pooled_ref[...] / rd
    lg = jnp.log(rd)
    h = jnp.dot(ave, w1a_ref[...], preferred_element_type=jnp.float32)
    h = h + lg * w1b_ref[...] + b1_ref[...]
    h = jnp.maximum(h, 0.0)
    h = jnp.dot(h, w2_ref[...], preferred_element_type=jnp.float32)
    h = jnp.maximum(h + b2_ref[...], 0.0)
    mu = jnp.dot(h, wmu_ref[...], preferred_element_type=jnp.float32)
    mu_ref[...] = mu * smu_ref[...] + cmu_ref[...]
    lv = jnp.dot(h, wlv_ref[...], preferred_element_type=jnp.float32)
    lv = lv * slv_ref[...] + clv_ref[...]
    sc_ref[...] = jnp.maximum(lv, 0.0) + jnp.log1p(jnp.exp(-jnp.abs(lv)))


def _tc_mlp(pooled, read_depth, w1a, w1b, b1, w2, b2,
            wmu, cmu, smu, wlv, clv, slv):
    nrows = pooled.shape[0]
    blk = 512
    grid = (nrows // blk,)
    full = lambda shape: pl.BlockSpec(shape, lambda i: (0, 0))
    out_shape = (
        jax.ShapeDtypeStruct((nrows, T), jnp.float32),
        jax.ShapeDtypeStruct((nrows, T), jnp.float32),
    )
    return pl.pallas_call(
        _mlp_body,
        grid=grid,
        in_specs=[
            pl.BlockSpec((blk, H), lambda i: (i, 0)),
            pl.BlockSpec((blk, 1), lambda i: (i, 0)),
            full((H, H)), full((1, H)), full((1, H)),
            full((H, H)), full((1, H)),
            full((H, T)), full((1, T)), full((1, T)),
            full((H, T)), full((1, T)), full((1, T)),
        ],
        out_specs=(
            pl.BlockSpec((blk, T), lambda i: (i, 0)),
            pl.BlockSpec((blk, T), lambda i: (i, 0)),
        ),
        out_shape=out_shape,
    )(pooled, read_depth, w1a, w1b, b1, w2, b2,
      wmu, cmu, smu, wlv, clv, slv)


@jax.jit
def kernel(idx, read_depth, emb, fc1_w, fc1_b, fc2_w, fc2_b,
           fcmu_w, fcmu_b, fclv_w, fclv_b,
           bnmu_g, bnmu_b, bnlv_g, bnlv_b):
    idx3 = idx.reshape(B, 2, _LHALF).astype(jnp.int32)

    w1a = fc1_w[:, :H].T
    w1b = fc1_w[:, H].reshape(1, H)
    b1 = fc1_b.reshape(1, H)
    w2 = fc2_w.T
    b2 = fc2_b.reshape(1, H)
    bn_scale = 1.0 / jnp.sqrt(1.0 + 1e-5)
    smu = (bnmu_g * bn_scale).reshape(1, T)
    cmu = (fcmu_b.reshape(1, T) * smu) + bnmu_b.reshape(1, T)
    slv = (bnlv_g * bn_scale).reshape(1, T)
    clv = (fclv_b.reshape(1, T) * slv) + bnlv_b.reshape(1, T)
    wmu = fcmu_w.T
    wlv = fclv_w.T
    nchunks = 2
    cb = B // nchunks
    locs, scales = [], []
    for c in range(nchunks):
        pooled = _sc_pool(emb, idx3[c * cb:(c + 1) * cb])
        tl, ts = _tc_mlp(
            pooled, read_depth[c * cb:(c + 1) * cb],
            w1a, w1b, b1, w2, b2, wmu, cmu, smu, wlv, clv, slv)
        locs.append(tl)
        scales.append(ts)
    return (jnp.concatenate(locs), jnp.concatenate(scales))


# speedup vs baseline: 1.0298x; 1.0298x over previous
import functools

import jax
import jax.numpy as jnp
from jax import lax
from jax.experimental import pallas as pl
from jax.experimental.pallas import tpu as pltpu
from jax.experimental.pallas import tpu_sc as plsc

V = 1000000
H = 128
T = 32
B = 4096
L = 200

_NC = 2
_NS = 16
_NW = _NC * _NS
_BPW = B // _NW
_LHALF = L // 2


_NB = 4
_NI = 8


def _pool_body(nrows, emb_hbm, idx_hbm, out_hbm, idx_v, rows_v, acc_v,
               sem0, sem1, sem2, sem3, isem):
    _BPW = nrows // _NW
    wid = lax.axis_index("s") * _NC + lax.axis_index("c")
    base = wid * _BPW

    sems = (sem0, sem1, sem2, sem3)

    def issue(b, buf, islot):
        sem = sems[buf]
        for j in range(2):
            pltpu.async_copy(
                emb_hbm.at[idx_v.at[islot, j]],
                rows_v.at[buf, j], sem)

    def drain(buf):
        pltpu.make_async_copy(
            emb_hbm.at[pl.ds(0, L)], rows_v.at[buf], sems[buf]).wait()

    def drain_idx(islot):
        pltpu.make_async_copy(
            idx_hbm.at[0], idx_v.at[islot], isem).wait()

    def accumulate(b, buf):
        def acc_body(r, accs):
            return tuple(
                accs[c]
                + rows_v[buf, 0, r, pl.ds(c * 16, 16)]
                + rows_v[buf, 1, r, pl.ds(c * 16, 16)]
                for c in range(H // 16)
            )

        accs = lax.fori_loop(
            0, _LHALF, acc_body,
            tuple(jnp.zeros((16,), jnp.float32) for _ in range(H // 16)),
            unroll=4,
        )
        for c in range(H // 16):
            acc_v[b, pl.ds(c * 16, 16)] = accs[c]

    for k in range(_NB):
        pltpu.sync_copy(idx_hbm.at[base + k], idx_v.at[k])
        issue(k, k, k)
    for k in range(_NB, _NI):
        pltpu.async_copy(idx_hbm.at[base + k], idx_v.at[k], isem)

    def quad_body(i, _):
        b = _NB * i
        for k in range(_NB):
            r = b + k
            drain(k)
            accumulate(r, k)

            @pl.when(r + _NB < _BPW)
            def _():
                drain_idx((r + _NB) % _NI)
                issue(r + _NB, k, (r + _NB) % _NI)

            @pl.when(r + _NI < _BPW)
            def _():
                pltpu.async_copy(
                    idx_hbm.at[base + r + _NI],
                    idx_v.at[(r + _NI) % _NI], isem)
        return 0

    lax.fori_loop(0, _BPW // _NB, quad_body, 0)

    pltpu.sync_copy(acc_v, out_hbm.at[pl.ds(base, _BPW)])


def _sc_pool(emb, idx3):
    nrows = idx3.shape[0]
    mesh = plsc.VectorSubcoreMesh(
        core_axis_name="c", subcore_axis_name="s",
        num_cores=_NC, num_subcores=_NS)
    k = pl.kernel(
        functools.partial(_pool_body, nrows),
        out_type=jax.ShapeDtypeStruct((nrows, H), jnp.float32),
        mesh=mesh,
        scratch_types=[
            pltpu.VMEM((_NI, 2, _LHALF), jnp.int32),
            pltpu.VMEM((_NB, 2, _LHALF, H), jnp.float32),
            pltpu.VMEM((nrows // _NW, H), jnp.float32),
            pltpu.SemaphoreType.DMA,
            pltpu.SemaphoreType.DMA,
            pltpu.SemaphoreType.DMA,
            pltpu.SemaphoreType.DMA,
            pltpu.SemaphoreType.DMA,
        ],
    )
    return k(emb, idx3)


def _mlp_body(pooled_ref, rd_ref, w1a_ref, w1b_ref, b1_ref, w2_ref, b2_ref,
              wmu_ref, cmu_ref, smu_ref, wlv_ref, clv_ref, slv_ref,
              mu_ref, sc_ref):
    rd = rd_ref[...]
    ave = pooled_ref[...] / rd
    lg = jnp.log(rd)
    h = jnp.dot(ave, w1a_ref[...], preferred_element_type=jnp.float32)
    h = h + lg * w1b_ref[...] + b1_ref[...]
    h = jnp.maximum(h, 0.0)
    h = jnp.dot(h, w2_ref[...], preferred_element_type=jnp.float32)
    h = jnp.maximum(h + b2_ref[...], 0.0)
    mu = jnp.dot(h, wmu_ref[...], preferred_element_type=jnp.float32)
    mu_ref[...] = mu * smu_ref[...] + cmu_ref[...]
    lv = jnp.dot(h, wlv_ref[...], preferred_element_type=jnp.float32)
    lv = lv * slv_ref[...] + clv_ref[...]
    sc_ref[...] = jnp.maximum(lv, 0.0) + jnp.log1p(jnp.exp(-jnp.abs(lv)))


def _tc_mlp(pooled, read_depth, w1a, w1b, b1, w2, b2,
            wmu, cmu, smu, wlv, clv, slv):
    nrows = pooled.shape[0]
    blk = 512
    grid = (nrows // blk,)
    full = lambda shape: pl.BlockSpec(shape, lambda i: (0, 0))
    out_shape = (
        jax.ShapeDtypeStruct((nrows, T), jnp.float32),
        jax.ShapeDtypeStruct((nrows, T), jnp.float32),
    )
    return pl.pallas_call(
        _mlp_body,
        grid=grid,
        in_specs=[
            pl.BlockSpec((blk, H), lambda i: (i, 0)),
            pl.BlockSpec((blk, 1), lambda i: (i, 0)),
            full((H, H)), full((1, H)), full((1, H)),
            full((H, H)), full((1, H)),
            full((H, T)), full((1, T)), full((1, T)),
            full((H, T)), full((1, T)), full((1, T)),
        ],
        out_specs=(
            pl.BlockSpec((blk, T), lambda i: (i, 0)),
            pl.BlockSpec((blk, T), lambda i: (i, 0)),
        ),
        out_shape=out_shape,
    )(pooled, read_depth, w1a, w1b, b1, w2, b2,
      wmu, cmu, smu, wlv, clv, slv)


@jax.jit
def kernel(idx, read_depth, emb, fc1_w, fc1_b, fc2_w, fc2_b,
           fcmu_w, fcmu_b, fclv_w, fclv_b,
           bnmu_g, bnmu_b, bnlv_g, bnlv_b):
    idx3 = idx.reshape(B, 2, _LHALF).astype(jnp.int32)

    w1a = fc1_w[:, :H].T
    w1b = fc1_w[:, H].reshape(1, H)
    b1 = fc1_b.reshape(1, H)
    w2 = fc2_w.T
    b2 = fc2_b.reshape(1, H)
    bn_scale = 1.0 / jnp.sqrt(1.0 + 1e-5)
    smu = (bnmu_g * bn_scale).reshape(1, T)
    cmu = (fcmu_b.reshape(1, T) * smu) + bnmu_b.reshape(1, T)
    slv = (bnlv_g * bn_scale).reshape(1, T)
    clv = (fclv_b.reshape(1, T) * slv) + bnlv_b.reshape(1, T)
    wmu = fcmu_w.T
    wlv = fclv_w.T
    nchunks = 1
    cb = B // nchunks
    locs, scales = [], []
    for c in range(nchunks):
        pooled = _sc_pool(emb, idx3[c * cb:(c + 1) * cb])
        tl, ts = _tc_mlp(
            pooled, read_depth[c * cb:(c + 1) * cb],
            w1a, w1b, b1, w2, b2, wmu, cmu, smu, wlv, clv, slv)
        locs.append(tl)
        scales.append(ts)
    return (jnp.concatenate(locs), jnp.concatenate(scales))


# speedup vs baseline: 1.0344x; 1.0045x over previous
import functools

import jax
import jax.numpy as jnp
from jax import lax
from jax.experimental import pallas as pl
from jax.experimental.pallas import tpu as pltpu
from jax.experimental.pallas import tpu_sc as plsc

V = 1000000
H = 128
T = 32
B = 4096
L = 200

_NC = 2
_NS = 16
_NW = _NC * _NS
_BPW = B // _NW
_LHALF = L // 2


_NB = 4
_NI = 8


def _pool_body(nrows, emb_hbm, idx_hbm, out_hbm, idx_v, rows_v, acc_v,
               sem0, sem1, sem2, sem3, isem):
    _BPW = nrows // _NW
    wid = lax.axis_index("s") * _NC + lax.axis_index("c")
    base = wid * _BPW

    sems = (sem0, sem1, sem2, sem3)

    def issue(b, buf, islot):
        sem = sems[buf]
        for j in range(2):
            pltpu.async_copy(
                emb_hbm.at[idx_v.at[islot, j]],
                rows_v.at[buf, j], sem)

    def drain(buf):
        pltpu.make_async_copy(
            emb_hbm.at[pl.ds(0, L)], rows_v.at[buf], sems[buf]).wait()

    def drain_idx(islot):
        pltpu.make_async_copy(
            idx_hbm.at[0], idx_v.at[islot], isem).wait()

    def accumulate(b, buf):
        def acc_body(r, accs):
            return tuple(
                accs[c]
                + rows_v[buf, 0, r, pl.ds(c * 16, 16)]
                + rows_v[buf, 1, r, pl.ds(c * 16, 16)]
                for c in range(H // 16)
            )

        accs = lax.fori_loop(
            0, _LHALF, acc_body,
            tuple(jnp.zeros((16,), jnp.float32) for _ in range(H // 16)),
            unroll=8,
        )
        for c in range(H // 16):
            acc_v[b, pl.ds(c * 16, 16)] = accs[c]

    for k in range(_NB):
        pltpu.sync_copy(idx_hbm.at[base + k], idx_v.at[k])
        issue(k, k, k)
    for k in range(_NB, _NI):
        pltpu.async_copy(idx_hbm.at[base + k], idx_v.at[k], isem)

    def quad_body(i, _):
        b = _NB * i
        for k in range(_NB):
            r = b + k
            drain(k)
            accumulate(r, k)

            @pl.when(r + _NB < _BPW)
            def _():
                drain_idx((r + _NB) % _NI)
                issue(r + _NB, k, (r + _NB) % _NI)

            @pl.when(r + _NI < _BPW)
            def _():
                pltpu.async_copy(
                    idx_hbm.at[base + r + _NI],
                    idx_v.at[(r + _NI) % _NI], isem)
        return 0

    lax.fori_loop(0, _BPW // _NB, quad_body, 0)

    pltpu.sync_copy(acc_v, out_hbm.at[pl.ds(base, _BPW)])


def _sc_pool(emb, idx3):
    nrows = idx3.shape[0]
    mesh = plsc.VectorSubcoreMesh(
        core_axis_name="c", subcore_axis_name="s",
        num_cores=_NC, num_subcores=_NS)
    k = pl.kernel(
        functools.partial(_pool_body, nrows),
        out_type=jax.ShapeDtypeStruct((nrows, H), jnp.float32),
        mesh=mesh,
        scratch_types=[
            pltpu.VMEM((_NI, 2, _LHALF), jnp.int32),
            pltpu.VMEM((_NB, 2, _LHALF, H), jnp.float32),
            pltpu.VMEM((nrows // _NW, H), jnp.float32),
            pltpu.SemaphoreType.DMA,
            pltpu.SemaphoreType.DMA,
            pltpu.SemaphoreType.DMA,
            pltpu.SemaphoreType.DMA,
            pltpu.SemaphoreType.DMA,
        ],
    )
    return k(emb, idx3)


def _mlp_body(pooled_ref, rd_ref, w1a_ref, w1b_ref, b1_ref, w2_ref, b2_ref,
              wmu_ref, cmu_ref, smu_ref, wlv_ref, clv_ref, slv_ref,
              mu_ref, sc_ref):
    rd = rd_ref[...]
    ave = pooled_ref[...] / rd
    lg = jnp.log(rd)
    h = jnp.dot(ave, w1a_ref[...], preferred_element_type=jnp.float32)
    h = h + lg * w1b_ref[...] + b1_ref[...]
    h = jnp.maximum(h, 0.0)
    h = jnp.dot(h, w2_ref[...], preferred_element_type=jnp.float32)
    h = jnp.maximum(h + b2_ref[...], 0.0)
    mu = jnp.dot(h, wmu_ref[...], preferred_element_type=jnp.float32)
    mu_ref[...] = mu * smu_ref[...] + cmu_ref[...]
    lv = jnp.dot(h, wlv_ref[...], preferred_element_type=jnp.float32)
    lv = lv * slv_ref[...] + clv_ref[...]
    sc_ref[...] = jnp.maximum(lv, 0.0) + jnp.log1p(jnp.exp(-jnp.abs(lv)))


def _tc_mlp(pooled, read_depth, w1a, w1b, b1, w2, b2,
            wmu, cmu, smu, wlv, clv, slv):
    nrows = pooled.shape[0]
    blk = 512
    grid = (nrows // blk,)
    full = lambda shape: pl.BlockSpec(shape, lambda i: (0, 0))
    out_shape = (
        jax.ShapeDtypeStruct((nrows, T), jnp.float32),
        jax.ShapeDtypeStruct((nrows, T), jnp.float32),
    )
    return pl.pallas_call(
        _mlp_body,
        grid=grid,
        in_specs=[
            pl.BlockSpec((blk, H), lambda i: (i, 0)),
            pl.BlockSpec((blk, 1), lambda i: (i, 0)),
            full((H, H)), full((1, H)), full((1, H)),
            full((H, H)), full((1, H)),
            full((H, T)), full((1, T)), full((1, T)),
            full((H, T)), full((1, T)), full((1, T)),
        ],
        out_specs=(
            pl.BlockSpec((blk, T), lambda i: (i, 0)),
            pl.BlockSpec((blk, T), lambda i: (i, 0)),
        ),
        out_shape=out_shape,
    )(pooled, read_depth, w1a, w1b, b1, w2, b2,
      wmu, cmu, smu, wlv, clv, slv)


@jax.jit
def kernel(idx, read_depth, emb, fc1_w, fc1_b, fc2_w, fc2_b,
           fcmu_w, fcmu_b, fclv_w, fclv_b,
           bnmu_g, bnmu_b, bnlv_g, bnlv_b):
    idx3 = idx.reshape(B, 2, _LHALF).astype(jnp.int32)

    w1a = fc1_w[:, :H].T
    w1b = fc1_w[:, H].reshape(1, H)
    b1 = fc1_b.reshape(1, H)
    w2 = fc2_w.T
    b2 = fc2_b.reshape(1, H)
    bn_scale = 1.0 / jnp.sqrt(1.0 + 1e-5)
    smu = (bnmu_g * bn_scale).reshape(1, T)
    cmu = (fcmu_b.reshape(1, T) * smu) + bnmu_b.reshape(1, T)
    slv = (bnlv_g * bn_scale).reshape(1, T)
    clv = (fclv_b.reshape(1, T) * slv) + bnlv_b.reshape(1, T)
    wmu = fcmu_w.T
    wlv = fclv_w.T
    nchunks = 1
    cb = B // nchunks
    locs, scales = [], []
    for c in range(nchunks):
        pooled = _sc_pool(emb, idx3[c * cb:(c + 1) * cb])
        tl, ts = _tc_mlp(
            pooled, read_depth[c * cb:(c + 1) * cb],
            w1a, w1b, b1, w2, b2, wmu, cmu, smu, wlv, clv, slv)
        locs.append(tl)
        scales.append(ts)
    return (jnp.concatenate(locs), jnp.concatenate(scales))
